# Initial kernel scaffold; baseline (speedup 1.0000x reference)
import jax
import jax.numpy as jnp
from jax.experimental import pallas as pl


def kernel(x, edge_index, batch, Wp, bp, Wq, bq, Wk, bk, Wv, bv, Ws, bs):
    raise NotImplementedError("write your pallas kernel here")


# speedup vs baseline: 3.6548x; 3.6548x over previous
import functools

import jax
import jax.numpy as jnp
from jax import lax
from jax.experimental import pallas as pl
from jax.experimental.pallas import tpu as pltpu
from jax.experimental.pallas import tpu_sc as plsc

N = 10000
E = 320000
B = 64
D_IN = 78
D = 128
L = 3

SCALE = 1.0 / (D ** 0.5)

NP = 10240
NC = 2
NS = 16
NW = NC * NS
EPW = E // NW
CB1 = 400
NBLK1 = EPW // CB1
CB2 = 80
NBLK2 = EPW // CB2
RPT = NP // NS
FCH = 80

RB = 128
NB = NP // RB
OH = NP // RB

_SC_PARAMS = pltpu.CompilerParams(needs_layout_passes=False)
_MESH = plsc.VectorSubcoreMesh(core_axis_name="c", subcore_axis_name="s")


def _qkvs(h, wq, bq, wk, bk, wv, bv, ws, bs):
    q = (jnp.dot(h, wq, preferred_element_type=jnp.float32) + bq) * SCALE
    k = jnp.dot(h, wk, preferred_element_type=jnp.float32) + bk
    v = jnp.dot(h, wv, preferred_element_type=jnp.float32) + bv
    s = jnp.dot(h, ws, preferred_element_type=jnp.float32) + bs
    return q, k, v, s


def _tc_first_body(x_ref, wp_ref, bp_ref, wq_ref, bq_ref, wk_ref, bk_ref,
                   wv_ref, bv_ref, ws_ref, bs_ref,
                   q_ref, k_ref, v_ref, s_ref):
    h = jnp.dot(x_ref[...], wp_ref[...],
                preferred_element_type=jnp.float32) + bp_ref[...]
    q, k, v, s = _qkvs(h, wq_ref[...], bq_ref[...], wk_ref[...], bk_ref[...],
                       wv_ref[...], bv_ref[...], ws_ref[...], bs_ref[...])
    q_ref[...] = q
    k_ref[...] = k
    v_ref[...] = v
    s_ref[...] = s


def _tc_first(x, wp, bp, wq, bq, wk, bk, wv, bv, ws, bs):
    full = lambda shape: pl.BlockSpec(shape, lambda j: (0, 0))
    return pl.pallas_call(
        _tc_first_body,
        grid=(NB,),
        in_specs=[
            pl.BlockSpec((RB, D_IN), lambda j: (j, 0)),
            full((D_IN, D)), full((1, D)),
            full((D, D)), full((1, D)),
            full((D, D)), full((1, D)),
            full((D, D)), full((1, D)),
            full((D, D)), full((1, D)),
        ],
        out_specs=[pl.BlockSpec((RB, D), lambda j: (j, 0))] * 4,
        out_shape=[jax.ShapeDtypeStruct((NP, D), jnp.float32)] * 4,
    )(x, wp, bp, wq, bq, wk, bk, wv, bv, ws, bs)


def _leaky_h(o0, o1, dp, sp):
    den = jnp.maximum(jnp.sum(dp, axis=0), 1e-30)
    h = (o0 + o1) / den[:, None] + sp
    return jnp.where(h >= 0, h, 0.01 * h)


def _tc_mid_body(o0_ref, o1_ref, dp_ref, sp_ref, wq_ref, bq_ref, wk_ref,
                 bk_ref, wv_ref, bv_ref, ws_ref, bs_ref,
                 q_ref, k_ref, v_ref, s_ref):
    h = _leaky_h(o0_ref[...], o1_ref[...], dp_ref[...], sp_ref[...])
    q, k, v, s = _qkvs(h, wq_ref[...], bq_ref[...], wk_ref[...], bk_ref[...],
                       wv_ref[...], bv_ref[...], ws_ref[...], bs_ref[...])
    q_ref[...] = q
    k_ref[...] = k
    v_ref[...] = v
    s_ref[...] = s


def _tc_mid(outp, dpart, sp, wq, bq, wk, bk, wv, bv, ws, bs):
    full = lambda shape: pl.BlockSpec(shape, lambda j: (0, 0))
    return pl.pallas_call(
        _tc_mid_body,
        grid=(NB,),
        in_specs=[
            pl.BlockSpec((RB, D), lambda j: (j, 0)),
            pl.BlockSpec((RB, D), lambda j: (j + OH, 0)),
            pl.BlockSpec((NW, RB), lambda j: (0, j)),
            pl.BlockSpec((RB, D), lambda j: (j, 0)),
            full((D, D)), full((1, D)),
            full((D, D)), full((1, D)),
            full((D, D)), full((1, D)),
            full((D, D)), full((1, D)),
        ],
        out_specs=[pl.BlockSpec((RB, D), lambda j: (j, 0))] * 4,
        out_shape=[jax.ShapeDtypeStruct((NP, D), jnp.float32)] * 4,
    )(outp, outp, dpart, sp, wq, bq, wk, bk, wv, bv, ws, bs)


def _tc_pool_body(o0_ref, o1_ref, dp_ref, sp_ref, b_ref, out_ref,
                  acc_ref, cnt_ref):
    j = pl.program_id(0)
    h = _leaky_h(o0_ref[...], o1_ref[...], dp_ref[...], sp_ref[...])
    bvec = b_ref[0, 0, :]
    onehot = (bvec[:, None]
              == lax.broadcasted_iota(jnp.int32, (RB, B), 1)
              ).astype(jnp.float32)

    @pl.when(j == 0)
    def _():
        acc_ref[...] = jnp.zeros((B, D), jnp.float32)
        cnt_ref[...] = jnp.zeros((B, D), jnp.float32)

    acc_ref[...] += lax.dot_general(
        onehot, h, (((0,), (0,)), ((), ())),
        preferred_element_type=jnp.float32)
    cnt_ref[...] += jnp.broadcast_to(
        jnp.sum(onehot, axis=0)[:, None], (B, D))

    @pl.when(j == NB - 1)
    def _():
        out_ref[...] = acc_ref[...] / jnp.maximum(cnt_ref[...], 1.0)


def _tc_pool(outp, dpart, sp, batch3):
    return pl.pallas_call(
        _tc_pool_body,
        grid=(NB,),
        in_specs=[
            pl.BlockSpec((RB, D), lambda j: (j, 0)),
            pl.BlockSpec((RB, D), lambda j: (j + OH, 0)),
            pl.BlockSpec((NW, RB), lambda j: (0, j)),
            pl.BlockSpec((RB, D), lambda j: (j, 0)),
            pl.BlockSpec((1, 1, RB), lambda j: (j, 0, 0)),
        ],
        out_specs=pl.BlockSpec((B, D), lambda j: (0, 0)),
        out_shape=jax.ShapeDtypeStruct((B, D), jnp.float32),
        scratch_shapes=[
            pltpu.VMEM((B, D), jnp.float32),
            pltpu.VMEM((B, D), jnp.float32),
        ],
    )(outp, outp, dpart, sp, batch3)


def _sc_alpha_body(q_hbm, k_hbm, src_hbm, dst_hbm, ex_hbm, dpart_hbm,
                   src_v, dst_v, qrows_v, krows_v, ex_v, den_v, sem0, sem1):
    c = lax.axis_index("c")
    s = lax.axis_index("s")
    wid = s * NC + c
    zero16 = jnp.zeros((16,), jnp.float32)
    iota16 = lax.iota(jnp.int32, 16)

    def zrow(i, _):
        den_v[pl.ds(i * 16, 16)] = zero16
        return 0
    lax.fori_loop(0, NP // 16, zrow, 0)

    ebase = wid * EPW

    def blk(b, _):
        base = ebase + b * CB1
        pltpu.sync_copy(src_hbm.at[pl.ds(base, CB1)], src_v)
        pltpu.sync_copy(dst_hbm.at[pl.ds(base, CB1)], dst_v)
        cq = pltpu.async_copy(q_hbm.at[dst_v], qrows_v, sem0)
        ck = pltpu.async_copy(k_hbm.at[src_v], krows_v, sem1)
        cq.wait()
        ck.wait()

        def grp(g, _2):
            r0 = g * 16
            row16 = iota16 + r0
            dst16 = dst_v[pl.ds(r0, 16)]

            def dotstep(dd, acc):
                col = jnp.full((16,), dd, jnp.int32)
                qv = plsc.load_gather(qrows_v, [row16, col])
                kv = plsc.load_gather(krows_v, [row16, col])
                return acc + qv * kv
            acc = lax.fori_loop(0, D, dotstep, zero16)
            exv = jnp.exp(acc)
            ex_v[pl.ds(r0, 16)] = exv
            plsc.addupdate_scatter(den_v, [dst16], exv)
            return 0
        lax.fori_loop(0, CB1 // 16, grp, 0)
        pltpu.sync_copy(ex_v, ex_hbm.at[pl.ds(base, CB1)])
        return 0
    lax.fori_loop(0, NBLK1, blk, 0)

    pltpu.sync_copy(den_v, dpart_hbm.at[wid])


def _sc_alpha(q, k, src, dst):
    run = functools.partial(
        pl.kernel,
        out_type=(jax.ShapeDtypeStruct((E,), jnp.float32),
                  jax.ShapeDtypeStruct((NW, NP), jnp.float32)),
        mesh=_MESH,
        compiler_params=_SC_PARAMS,
        scratch_types=[
            pltpu.VMEM((CB1,), jnp.int32),
            pltpu.VMEM((CB1,), jnp.int32),
            pltpu.VMEM((CB1, D), jnp.float32),
            pltpu.VMEM((CB1, D), jnp.float32),
            pltpu.VMEM((CB1,), jnp.float32),
            pltpu.VMEM((NP,), jnp.float32),
            pltpu.SemaphoreType.DMA,
            pltpu.SemaphoreType.DMA,
        ],
    )(_sc_alpha_body)
    return run(q, k, src, dst)


def _sc_agg_body(v_hbm, ex_hbm, src_hbm, dst_hbm, zeros_hbm, outp_hbm,
                 src_v, dst_v, ex_v, vrows_v, spmem_out, sem0):
    c = lax.axis_index("c")
    s = lax.axis_index("s")
    wid = s * NC + c

    pltpu.sync_copy(zeros_hbm, spmem_out.at[pl.ds(s * RPT, RPT)])
    plsc.subcore_barrier()

    ebase = wid * EPW

    def blk(b, _):
        base = ebase + b * CB2
        pltpu.sync_copy(src_hbm.at[pl.ds(base, CB2)], src_v)
        pltpu.sync_copy(dst_hbm.at[pl.ds(base, CB2)], dst_v)
        pltpu.sync_copy(ex_hbm.at[pl.ds(base, CB2)], ex_v)
        pltpu.async_copy(v_hbm.at[src_v], vrows_v, sem0).wait()

        def scale(e, _2):
            sp = plsc.load_gather(ex_v, [jnp.full((16,), e, jnp.int32)])
            for cc in range(D // 16):
                vrows_v[e, pl.ds(cc * 16, 16)] = (
                    vrows_v[e, pl.ds(cc * 16, 16)] * sp)
            return 0
        lax.fori_loop(0, CB2, scale, 0)

        pltpu.sync_copy(vrows_v, spmem_out.at[dst_v], add=True)
        return 0
    lax.fori_loop(0, NBLK2, blk, 0)

    plsc.subcore_barrier()

    def flush(t, _):
        r0 = s * RPT + t * FCH
        pltpu.sync_copy(spmem_out.at[pl.ds(r0, FCH)], vrows_v)
        pltpu.sync_copy(vrows_v, outp_hbm.at[pl.ds(c * NP + r0, FCH)])
        return 0
    lax.fori_loop(0, RPT // FCH, flush, 0)


def _sc_agg(v, ex, src, dst, zeros):
    run = functools.partial(
        pl.kernel,
        out_type=jax.ShapeDtypeStruct((NC * NP, D), jnp.float32),
        mesh=_MESH,
        compiler_params=_SC_PARAMS,
        scratch_types=[
            pltpu.VMEM((CB2,), jnp.int32),
            pltpu.VMEM((CB2,), jnp.int32),
            pltpu.VMEM((CB2,), jnp.float32),
            pltpu.VMEM((CB2, D), jnp.float32),
            pltpu.VMEM_SHARED((NP, D), jnp.float32),
            pltpu.SemaphoreType.DMA,
        ],
    )(_sc_agg_body)
    return run(v, ex, src, dst, zeros)


def kernel(x, edge_index, batch, Wp, bp, Wq, bq, Wk, bk, Wv, bv, Ws, bs):
    src = edge_index[0]
    dst = edge_index[1]
    xp = jnp.pad(x, ((0, NP - N), (0, 0)))
    batch_p = jnp.concatenate(
        [batch, jnp.full((NP - N,), B, jnp.int32)]).reshape(NB, 1, RB)
    zeros = jnp.zeros((RPT, D), jnp.float32)
    b2 = lambda b: b.reshape(1, D)

    q, k, v, s = _tc_first(xp, Wp, b2(bp),
                           Wq[0], b2(bq[0]), Wk[0], b2(bk[0]),
                           Wv[0], b2(bv[0]), Ws[0], b2(bs[0]))
    for i in range(L):
        ex, dpart = _sc_alpha(q, k, src, dst)
        outp = _sc_agg(v, ex, src, dst, zeros)
        if i < L - 1:
            q, k, v, s = _tc_mid(outp, dpart, s,
                                 Wq[i + 1], b2(bq[i + 1]),
                                 Wk[i + 1], b2(bk[i + 1]),
                                 Wv[i + 1], b2(bv[i + 1]),
                                 Ws[i + 1], b2(bs[i + 1]))
    return _tc_pool(outp, dpart, s, batch_p)


# speedup vs baseline: 4.1088x; 1.1242x over previous
import functools

import jax
import jax.numpy as jnp
from jax import lax
from jax.experimental import pallas as pl
from jax.experimental.pallas import tpu as pltpu
from jax.experimental.pallas import tpu_sc as plsc

N = 10000
E = 320000
B = 64
D_IN = 78
D = 128
L = 3

SCALE = 1.0 / (D ** 0.5)

NP = 10240
NC = 2
NS = 16
NW = NC * NS
EPW = E // NW
CH1 = 2000
CB1 = 400
NCH1 = EPW // CH1
NBPC1 = CH1 // CB1
CH2 = 2000
CB2 = 200
NCH2 = EPW // CH2
NBPC2 = CH2 // CB2
RPT = NP // NS
FCH = 80

RB = 128
NB = NP // RB
OH = NP // RB

_SC_PARAMS = pltpu.CompilerParams(needs_layout_passes=False)
_MESH = plsc.VectorSubcoreMesh(core_axis_name="c", subcore_axis_name="s")


def _qkvs(h, wq, bq, wk, bk, wv, bv, ws, bs):
    q = (jnp.dot(h, wq, preferred_element_type=jnp.float32) + bq) * SCALE
    k = jnp.dot(h, wk, preferred_element_type=jnp.float32) + bk
    v = jnp.dot(h, wv, preferred_element_type=jnp.float32) + bv
    s = jnp.dot(h, ws, preferred_element_type=jnp.float32) + bs
    return q, k, v, s


def _tc_first_body(x_ref, wp_ref, bp_ref, wq_ref, bq_ref, wk_ref, bk_ref,
                   wv_ref, bv_ref, ws_ref, bs_ref,
                   q_ref, k_ref, v_ref, s_ref):
    h = jnp.dot(x_ref[...], wp_ref[...],
                preferred_element_type=jnp.float32) + bp_ref[...]
    q, k, v, s = _qkvs(h, wq_ref[...], bq_ref[...], wk_ref[...], bk_ref[...],
                       wv_ref[...], bv_ref[...], ws_ref[...], bs_ref[...])
    q_ref[...] = q
    k_ref[...] = k
    v_ref[...] = v
    s_ref[...] = s


def _tc_first(x, wp, bp, wq, bq, wk, bk, wv, bv, ws, bs):
    full = lambda shape: pl.BlockSpec(shape, lambda j: (0, 0))
    return pl.pallas_call(
        _tc_first_body,
        grid=(NB,),
        in_specs=[
            pl.BlockSpec((RB, D_IN), lambda j: (j, 0)),
            full((D_IN, D)), full((1, D)),
            full((D, D)), full((1, D)),
            full((D, D)), full((1, D)),
            full((D, D)), full((1, D)),
            full((D, D)), full((1, D)),
        ],
        out_specs=[pl.BlockSpec((RB, D), lambda j: (j, 0))] * 4,
        out_shape=[jax.ShapeDtypeStruct((NP, D), jnp.float32)] * 4,
    )(x, wp, bp, wq, bq, wk, bk, wv, bv, ws, bs)


def _leaky_h(o0, o1, dp, sp):
    den = jnp.maximum(jnp.sum(dp, axis=0), 1e-30)
    h = (o0 + o1) / den[:, None] + sp
    return jnp.where(h >= 0, h, 0.01 * h)


def _tc_mid_body(o0_ref, o1_ref, dp_ref, sp_ref, wq_ref, bq_ref, wk_ref,
                 bk_ref, wv_ref, bv_ref, ws_ref, bs_ref,
                 q_ref, k_ref, v_ref, s_ref):
    h = _leaky_h(o0_ref[...], o1_ref[...], dp_ref[...], sp_ref[...])
    q, k, v, s = _qkvs(h, wq_ref[...], bq_ref[...], wk_ref[...], bk_ref[...],
                       wv_ref[...], bv_ref[...], ws_ref[...], bs_ref[...])
    q_ref[...] = q
    k_ref[...] = k
    v_ref[...] = v
    s_ref[...] = s


def _tc_mid(outp, dpart, sp, wq, bq, wk, bk, wv, bv, ws, bs):
    full = lambda shape: pl.BlockSpec(shape, lambda j: (0, 0))
    return pl.pallas_call(
        _tc_mid_body,
        grid=(NB,),
        in_specs=[
            pl.BlockSpec((RB, D), lambda j: (j, 0)),
            pl.BlockSpec((RB, D), lambda j: (j + OH, 0)),
            pl.BlockSpec((NW, RB), lambda j: (0, j)),
            pl.BlockSpec((RB, D), lambda j: (j, 0)),
            full((D, D)), full((1, D)),
            full((D, D)), full((1, D)),
            full((D, D)), full((1, D)),
            full((D, D)), full((1, D)),
        ],
        out_specs=[pl.BlockSpec((RB, D), lambda j: (j, 0))] * 4,
        out_shape=[jax.ShapeDtypeStruct((NP, D), jnp.float32)] * 4,
    )(outp, outp, dpart, sp, wq, bq, wk, bk, wv, bv, ws, bs)


def _tc_pool_body(o0_ref, o1_ref, dp_ref, sp_ref, b_ref, out_ref,
                  acc_ref, cnt_ref):
    j = pl.program_id(0)
    h = _leaky_h(o0_ref[...], o1_ref[...], dp_ref[...], sp_ref[...])
    bvec = b_ref[0, 0, :]
    onehot = (bvec[:, None]
              == lax.broadcasted_iota(jnp.int32, (RB, B), 1)
              ).astype(jnp.float32)

    @pl.when(j == 0)
    def _():
        acc_ref[...] = jnp.zeros((B, D), jnp.float32)
        cnt_ref[...] = jnp.zeros((B, D), jnp.float32)

    acc_ref[...] += lax.dot_general(
        onehot, h, (((0,), (0,)), ((), ())),
        preferred_element_type=jnp.float32)
    cnt_ref[...] += jnp.broadcast_to(
        jnp.sum(onehot, axis=0)[:, None], (B, D))

    @pl.when(j == NB - 1)
    def _():
        out_ref[...] = acc_ref[...] / jnp.maximum(cnt_ref[...], 1.0)


def _tc_pool(outp, dpart, sp, batch3):
    return pl.pallas_call(
        _tc_pool_body,
        grid=(NB,),
        in_specs=[
            pl.BlockSpec((RB, D), lambda j: (j, 0)),
            pl.BlockSpec((RB, D), lambda j: (j + OH, 0)),
            pl.BlockSpec((NW, RB), lambda j: (0, j)),
            pl.BlockSpec((RB, D), lambda j: (j, 0)),
            pl.BlockSpec((1, 1, RB), lambda j: (j, 0, 0)),
        ],
        out_specs=pl.BlockSpec((B, D), lambda j: (0, 0)),
        out_shape=jax.ShapeDtypeStruct((B, D), jnp.float32),
        scratch_shapes=[
            pltpu.VMEM((B, D), jnp.float32),
            pltpu.VMEM((B, D), jnp.float32),
        ],
    )(outp, outp, dpart, sp, batch3)


def _sc_alpha_body(q_hbm, k_hbm, src_hbm, dst_hbm, ex_hbm, dpart_hbm,
                   src_v, dst_v, qrows_v, krows_v, ex_v, den_v, sem0, sem1):
    c = lax.axis_index("c")
    s = lax.axis_index("s")
    wid = s * NC + c
    zero16 = jnp.zeros((16,), jnp.float32)
    iota16 = lax.iota(jnp.int32, 16)

    def zrow(i, _):
        den_v[pl.ds(i * 16, 16)] = zero16
        return 0
    lax.fori_loop(0, NP // 16, zrow, 0)

    ebase = wid * EPW

    def chunk(ci, _):
        cbase = ebase + ci * CH1
        pltpu.sync_copy(src_hbm.at[pl.ds(cbase, CH1)], src_v)
        pltpu.sync_copy(dst_hbm.at[pl.ds(cbase, CH1)], dst_v)

        def blk(b, _1):
            e0 = b * CB1
            cq = pltpu.async_copy(
                q_hbm.at[dst_v.at[pl.ds(e0, CB1)]], qrows_v, sem0)
            ck = pltpu.async_copy(
                k_hbm.at[src_v.at[pl.ds(e0, CB1)]], krows_v, sem1)
            cq.wait()
            ck.wait()

            def grp(g, _2):
                r0 = g * 16
                row16 = iota16 + r0
                dst16 = dst_v[pl.ds(e0 + r0, 16)]
                acc = zero16
                for dd in range(D):
                    col = jnp.full((16,), dd, jnp.int32)
                    qv = plsc.load_gather(qrows_v, [row16, col])
                    kv = plsc.load_gather(krows_v, [row16, col])
                    acc = acc + qv * kv
                exv = jnp.exp(acc)
                ex_v[pl.ds(e0 + r0, 16)] = exv
                plsc.addupdate_scatter(den_v, [dst16], exv)
                return 0
            lax.fori_loop(0, CB1 // 16, grp, 0)
            return 0
        lax.fori_loop(0, NBPC1, blk, 0)
        pltpu.sync_copy(ex_v, ex_hbm.at[pl.ds(cbase, CH1)])
        return 0
    lax.fori_loop(0, NCH1, chunk, 0)

    pltpu.sync_copy(den_v, dpart_hbm.at[wid])


def _sc_alpha(q, k, src, dst):
    run = functools.partial(
        pl.kernel,
        out_type=(jax.ShapeDtypeStruct((E,), jnp.float32),
                  jax.ShapeDtypeStruct((NW, NP), jnp.float32)),
        mesh=_MESH,
        compiler_params=_SC_PARAMS,
        scratch_types=[
            pltpu.VMEM((CH1,), jnp.int32),
            pltpu.VMEM((CH1,), jnp.int32),
            pltpu.VMEM((CB1, D), jnp.float32),
            pltpu.VMEM((CB1, D), jnp.float32),
            pltpu.VMEM((CH1,), jnp.float32),
            pltpu.VMEM((NP,), jnp.float32),
            pltpu.SemaphoreType.DMA,
            pltpu.SemaphoreType.DMA,
        ],
    )(_sc_alpha_body)
    return run(q, k, src, dst)


def _sc_agg_body(v_hbm, ex_hbm, src_hbm, dst_hbm, zeros_hbm, outp_hbm,
                 src_v, dst_v, ex_v, vrows_v, spmem_out, sem0):
    c = lax.axis_index("c")
    s = lax.axis_index("s")
    wid = s * NC + c

    pltpu.sync_copy(zeros_hbm, spmem_out.at[pl.ds(s * RPT, RPT)])
    plsc.subcore_barrier()

    ebase = wid * EPW

    def chunk(ci, _):
        cbase = ebase + ci * CH2
        pltpu.sync_copy(src_hbm.at[pl.ds(cbase, CH2)], src_v)
        pltpu.sync_copy(ex_hbm.at[pl.ds(cbase, CH2)], ex_v)

        def blk(b, _1):
            e0 = b * CB2
            pltpu.sync_copy(dst_hbm.at[pl.ds(cbase + e0, CB2)], dst_v)
            pltpu.async_copy(
                v_hbm.at[src_v.at[pl.ds(e0, CB2)]], vrows_v, sem0).wait()

            def scale(e5, _2):
                for u in range(5):
                    e = e5 * 5 + u
                    sp = plsc.load_gather(
                        ex_v, [jnp.full((16,), e0 + e, jnp.int32)])
                    for cc in range(D // 16):
                        vrows_v[e, pl.ds(cc * 16, 16)] = (
                            vrows_v[e, pl.ds(cc * 16, 16)] * sp)
                return 0
            lax.fori_loop(0, CB2 // 5, scale, 0)

            pltpu.sync_copy(vrows_v, spmem_out.at[dst_v], add=True)
            return 0
        lax.fori_loop(0, NBPC2, blk, 0)
        return 0
    lax.fori_loop(0, NCH2, chunk, 0)

    plsc.subcore_barrier()

    def flush(t, _):
        r0 = s * RPT + t * FCH
        pltpu.sync_copy(spmem_out.at[pl.ds(r0, FCH)],
                        vrows_v.at[pl.ds(0, FCH)])
        pltpu.sync_copy(vrows_v.at[pl.ds(0, FCH)],
                        outp_hbm.at[pl.ds(c * NP + r0, FCH)])
        return 0
    lax.fori_loop(0, RPT // FCH, flush, 0)


def _sc_agg(v, ex, src, dst, zeros):
    run = functools.partial(
        pl.kernel,
        out_type=jax.ShapeDtypeStruct((NC * NP, D), jnp.float32),
        mesh=_MESH,
        compiler_params=_SC_PARAMS,
        scratch_types=[
            pltpu.VMEM((CH2,), jnp.int32),
            pltpu.VMEM((CB2,), jnp.int32),
            pltpu.VMEM((CH2,), jnp.float32),
            pltpu.VMEM((CB2, D), jnp.float32),
            pltpu.VMEM_SHARED((NP, D), jnp.float32),
            pltpu.SemaphoreType.DMA,
        ],
    )(_sc_agg_body)
    return run(v, ex, src, dst, zeros)


def kernel(x, edge_index, batch, Wp, bp, Wq, bq, Wk, bk, Wv, bv, Ws, bs):
    src = edge_index[0]
    dst = edge_index[1]
    xp = jnp.pad(x, ((0, NP - N), (0, 0)))
    batch_p = jnp.concatenate(
        [batch, jnp.full((NP - N,), B, jnp.int32)]).reshape(NB, 1, RB)
    zeros = jnp.zeros((RPT, D), jnp.float32)
    b2 = lambda b: b.reshape(1, D)

    q, k, v, s = _tc_first(xp, Wp, b2(bp),
                           Wq[0], b2(bq[0]), Wk[0], b2(bk[0]),
                           Wv[0], b2(bv[0]), Ws[0], b2(bs[0]))
    for i in range(L):
        ex, dpart = _sc_alpha(q, k, src, dst)
        outp = _sc_agg(v, ex, src, dst, zeros)
        if i < L - 1:
            q, k, v, s = _tc_mid(outp, dpart, s,
                                 Wq[i + 1], b2(bq[i + 1]),
                                 Wk[i + 1], b2(bk[i + 1]),
                                 Wv[i + 1], b2(bv[i + 1]),
                                 Ws[i + 1], b2(bs[i + 1]))
    return _tc_pool(outp, dpart, s, batch_p)
